# baseline (speedup 1.0000x reference)
import functools

import jax
import jax.numpy as jnp
from jax import lax
from jax.experimental import pallas as pl
from jax.experimental.pallas import tpu as pltpu
from jax.experimental.pallas import tpu_sc as plsc

E = 8
K = 2
D = 1024
F = 2048
N = 4096
P = N * K
TILE = 128
L_ROUTED = P + E * TILE
NUM_RTILES = L_ROUTED // TILE
NUM_STILES = N // TILE

_NC = 2
_NS = 16
_NW = _NC * _NS


def _router_body(x_ref, wr_ref, meta_ref):
    lt = lax.dot_general(wr_ref[...].astype(jnp.bfloat16),
                         x_ref[...].astype(jnp.bfloat16),
                         (((1,), (1,)), ((), ())),
                         preferred_element_type=jnp.float32)
    row = lax.broadcasted_iota(jnp.int32, (E, N), 0)
    l1 = jnp.max(lt, axis=0, keepdims=True)
    i1 = jnp.min(jnp.where(lt == l1, row, E), axis=0, keepdims=True)
    masked = jnp.where(row == i1, -jnp.inf, lt)
    l2 = jnp.max(masked, axis=0, keepdims=True)
    i2 = jnp.min(jnp.where(masked == l2, row, E), axis=0, keepdims=True)
    w1 = 1.0 / (1.0 + jnp.exp(l2 - l1))
    w2 = 1.0 / (1.0 + jnp.exp(l1 - l2))
    meta_ref[0:1, :] = i1.astype(jnp.float32)
    meta_ref[1:2, :] = i2.astype(jnp.float32)
    meta_ref[2:3, :] = w1
    meta_ref[3:4, :] = w2
    meta_ref[4:5, :] = jnp.zeros((1, N), jnp.float32)
    meta_ref[5:6, :] = jnp.zeros((1, N), jnp.float32)
    meta_ref[6:7, :] = jnp.zeros((1, N), jnp.float32)
    meta_ref[7:8, :] = jnp.zeros((1, N), jnp.float32)


def _router(xf, w_router, interpret=False):
    return pl.pallas_call(
        _router_body,
        out_shape=jax.ShapeDtypeStruct((8, N), jnp.float32),
        interpret=interpret,
    )(xf, w_router)


_ROWS_PER_W = L_ROUTED // _NW
_DCHUNK = 48
_DSTEPS = _ROWS_PER_W // _DCHUNK
_DNB = 2
_DLEAD = 1


def _dispatch_body(x_hbm, map_hbm, xg_hbm, idx_v, buf_v, gsems, wsems):
    wid = lax.axis_index("s") * _NC + lax.axis_index("c")
    wbase = wid * _ROWS_PER_W
    pltpu.sync_copy(map_hbm.at[pl.ds(wbase, _ROWS_PER_W)], idx_v)

    def gather_desc(c, fire):
        b = c % _DNB
        mk = pltpu.async_copy if fire else pltpu.make_async_copy
        return mk(x_hbm.at[idx_v.at[pl.ds(c * _DCHUNK, _DCHUNK)]],
                  buf_v.at[b], gsems.at[b])

    def write_desc(c, fire):
        b = c % _DNB
        mk = pltpu.async_copy if fire else pltpu.make_async_copy
        return mk(buf_v.at[b], xg_hbm.at[pl.ds(wbase + c * _DCHUNK,
                                               _DCHUNK)], wsems.at[b])

    for c in range(_DLEAD):
        gather_desc(c, True)
    for c in range(_DSTEPS):
        gather_desc(c, False).wait()
        write_desc(c, True)
        j = c + _DLEAD
        if j < _DSTEPS:
            if j - _DNB >= 0:
                write_desc(j - _DNB, False).wait()
            gather_desc(j, True)
    for c in range(_DSTEPS - _DNB, _DSTEPS):
        write_desc(c, False).wait()


def _dispatch(xf, src_map, interpret=False):
    mesh = plsc.VectorSubcoreMesh(core_axis_name="c", subcore_axis_name="s")
    f = functools.partial(
        pl.kernel,
        out_type=jax.ShapeDtypeStruct((L_ROUTED, D), jnp.float32),
        mesh=mesh,
        scratch_types=[
            pltpu.VMEM((_ROWS_PER_W,), jnp.int32),
            pltpu.VMEM((_DNB, _DCHUNK, D), jnp.float32),
            pltpu.SemaphoreType.DMA((_DNB,)),
            pltpu.SemaphoreType.DMA((_DNB,)),
        ],
        interpret=interpret,
    )(_dispatch_body)
    return f(xf, src_map)


def _silu(x):
    return x / (1.0 + jnp.exp(-x))


def _gemm_body(sched_ref, xg_ref, g_ref, u_ref, d_ref, out_ref):
    t = pl.program_id(0)
    te = sched_ref[t]
    row0 = t * TILE

    @pl.when(row0 < sched_ref[NUM_RTILES + te])
    def _():
        xb = xg_ref[...]
        hg = lax.dot_general(xb, g_ref[0], (((1,), (1,)), ((), ())),
                             preferred_element_type=jnp.float32)
        hu = lax.dot_general(xb, u_ref[0], (((1,), (1,)), ((), ())),
                             preferred_element_type=jnp.float32)
        h = _silu(hg) * hu
        out_ref[...] = lax.dot_general(h, d_ref[0], (((1,), (1,)), ((), ())),
                                       preferred_element_type=jnp.float32)


def _gemm(sched, xg, experts_gate, experts_up, experts_down, interpret=False):
    grid_spec = pltpu.PrefetchScalarGridSpec(
        num_scalar_prefetch=1,
        grid=(NUM_RTILES,),
        in_specs=[
            pl.BlockSpec((TILE, D), lambda t, s: (t, 0)),
            pl.BlockSpec((1, F, D), lambda t, s: (s[t], 0, 0)),
            pl.BlockSpec((1, F, D), lambda t, s: (s[t], 0, 0)),
            pl.BlockSpec((1, D, F), lambda t, s: (s[t], 0, 0)),
        ],
        out_specs=pl.BlockSpec((TILE, D), lambda t, s: (t, 0)),
    )
    return pl.pallas_call(
        _gemm_body,
        grid_spec=grid_spec,
        out_shape=jax.ShapeDtypeStruct((L_ROUTED, D), jnp.float32),
        interpret=interpret,
    )(sched, xg, experts_gate, experts_up, experts_down)


def _shared_body(x_ref, g_ref, u_ref, d_ref, out_ref):
    xb = x_ref[...]
    hg = lax.dot_general(xb, g_ref[...], (((1,), (1,)), ((), ())),
                         preferred_element_type=jnp.float32)
    hu = lax.dot_general(xb, u_ref[...], (((1,), (1,)), ((), ())),
                         preferred_element_type=jnp.float32)
    h = _silu(hg) * hu
    out_ref[...] = lax.dot_general(h, d_ref[...], (((1,), (1,)), ((), ())),
                                   preferred_element_type=jnp.float32)


def _shared(xf, g, u, d, interpret=False):
    return pl.pallas_call(
        _shared_body,
        grid=(NUM_STILES,),
        in_specs=[
            pl.BlockSpec((TILE, D), lambda t: (t, 0)),
            pl.BlockSpec((F, D), lambda t: (0, 0)),
            pl.BlockSpec((F, D), lambda t: (0, 0)),
            pl.BlockSpec((D, F), lambda t: (0, 0)),
        ],
        out_specs=pl.BlockSpec((TILE, D), lambda t: (t, 0)),
        out_shape=jax.ShapeDtypeStruct((N, D), jnp.float32),
        interpret=interpret,
    )(xf, g, u, d)


_TOK_PER_W = N // _NW
_CCHUNK = 8
_CSTEPS = _TOK_PER_W // _CCHUNK


def _combine_body(eout_hbm, es_hbm, pos1_hbm, pos2_hbm, w1_hbm, w2_hbm,
                  out_hbm, p1_v, p2_v, w1_v, w2_v, r1_v, r2_v, rs_v, out_v,
                  g1sems, g2sems, gssems, wsems):
    wid = lax.axis_index("s") * _NC + lax.axis_index("c")
    wbase = wid * _TOK_PER_W
    pltpu.sync_copy(pos1_hbm.at[pl.ds(wbase, _TOK_PER_W)], p1_v)
    pltpu.sync_copy(pos2_hbm.at[pl.ds(wbase, _TOK_PER_W)], p2_v)
    pltpu.sync_copy(w1_hbm.at[pl.ds(wbase, _TOK_PER_W)], w1_v)
    pltpu.sync_copy(w2_hbm.at[pl.ds(wbase, _TOK_PER_W)], w2_v)

    def gathers(b, c, fire):
        mk = pltpu.async_copy if fire else pltpu.make_async_copy
        sl = pl.ds(pl.multiple_of(c * _CCHUNK, _CCHUNK), _CCHUNK)
        lsl = pl.ds(pl.multiple_of(wbase + c * _CCHUNK, _CCHUNK), _CCHUNK)
        return (
            mk(eout_hbm.at[p1_v.at[sl]], r1_v.at[b], g1sems.at[b]),
            mk(eout_hbm.at[p2_v.at[sl]], r2_v.at[b], g2sems.at[b]),
            mk(es_hbm.at[lsl], rs_v.at[b], gssems.at[b]),
        )

    def write_desc(b, c, fire):
        mk = pltpu.async_copy if fire else pltpu.make_async_copy
        lsl = pl.ds(pl.multiple_of(wbase + c * _CCHUNK, _CCHUNK), _CCHUNK)
        return mk(out_v.at[b], out_hbm.at[lsl], wsems.at[b])

    for b in range(2):
        gathers(b, b, True)

    def outer(k, _):
        for b in range(2):
            c = 2 * k + b
            for h in gathers(b, 0, False):
                h.wait()

            @pl.when(k > 0)
            def _():
                write_desc(b, 0, False).wait()

            def tok(i, _):
                row = c * _CCHUNK + i
                w1b = w1_v[row, :]
                w2b = w2_v[row, :]
                for col in range(D // 16):
                    sl = pl.ds(col * 16, 16)
                    out_v[b, i, sl] = (r1_v[b, i, sl] * w1b
                                       + r2_v[b, i, sl] * w2b
                                       + rs_v[b, i, sl])
                return 0

            lax.fori_loop(0, _CCHUNK, tok, 0)
            write_desc(b, c, True)

            @pl.when(k < _CSTEPS // 2 - 1)
            def _():
                gathers(b, c + 2, True)
        return 0

    lax.fori_loop(0, _CSTEPS // 2, outer, 0)
    for b in range(2):
        write_desc(b, 0, False).wait()


def _combine(eout, eshared, pos1, pos2, w1, w2, interpret=False):
    mesh = plsc.VectorSubcoreMesh(core_axis_name="c", subcore_axis_name="s")
    f = functools.partial(
        pl.kernel,
        out_type=jax.ShapeDtypeStruct((N, D), jnp.float32),
        mesh=mesh,
        scratch_types=[
            pltpu.VMEM((_TOK_PER_W,), jnp.int32),
            pltpu.VMEM((_TOK_PER_W,), jnp.int32),
            pltpu.VMEM((_TOK_PER_W, 16), jnp.float32),
            pltpu.VMEM((_TOK_PER_W, 16), jnp.float32),
            pltpu.VMEM((2, _CCHUNK, D), jnp.float32),
            pltpu.VMEM((2, _CCHUNK, D), jnp.float32),
            pltpu.VMEM((2, _CCHUNK, D), jnp.float32),
            pltpu.VMEM((2, _CCHUNK, D), jnp.float32),
            pltpu.SemaphoreType.DMA((2,)),
            pltpu.SemaphoreType.DMA((2,)),
            pltpu.SemaphoreType.DMA((2,)),
            pltpu.SemaphoreType.DMA((2,)),
        ],
        interpret=interpret,
    )(_combine_body)
    w1b = jnp.broadcast_to(w1[:, None], (N, 16))
    w2b = jnp.broadcast_to(w2[:, None], (N, 16))
    return f(eout, eshared, pos1, pos2, w1b, w2b)


def kernel(x, w_router, shared_gate, shared_up, shared_down,
           experts_gate, experts_up, experts_down):
    xf = x.reshape(N, D)

    meta = _router(xf, w_router)
    e1 = meta[0].astype(jnp.int32)
    e2 = meta[1].astype(jnp.int32)
    w1 = meta[2]
    w2 = meta[3]

    pairs_e = jnp.stack([e1, e2], axis=1).reshape(-1)
    oh = (pairs_e[:, None] == jnp.arange(E)[None, :]).astype(jnp.int32)
    cum = jnp.cumsum(oh, axis=0)
    counts = cum[-1]
    rank = jnp.sum(oh * (cum - oh), axis=1)
    pc = ((counts + TILE - 1) // TILE) * TILE
    bounds = jnp.cumsum(pc)
    pstart = bounds - pc
    dst_pair = (jnp.sum(oh * pstart[None, :], axis=1) + rank).astype(jnp.int32)
    pos1 = dst_pair[0::2]
    pos2 = dst_pair[1::2]
    src_map = jnp.zeros(L_ROUTED, jnp.int32).at[dst_pair].set(
        jnp.arange(P, dtype=jnp.int32) // K)
    t_row0 = jnp.arange(NUM_RTILES, dtype=jnp.int32) * TILE
    tile_expert = jnp.minimum(
        jnp.sum((t_row0[:, None] >= bounds[None, :]).astype(jnp.int32),
                axis=1), E - 1)
    real_end = pstart + counts
    sched = jnp.concatenate([tile_expert, real_end])

    xg = _dispatch(xf, src_map)
    eshared = _shared(xf, shared_gate, shared_up, shared_down)
    eout = _gemm(sched, xg, experts_gate, experts_up, experts_down)
    out = _combine(eout, eshared, pos1, pos2, w1, w2)
    return out.reshape(x.shape)


# speedup vs baseline: 1.4333x; 1.4333x over previous
import functools

import jax
import jax.numpy as jnp
from jax import lax
from jax.experimental import pallas as pl
from jax.experimental.pallas import tpu as pltpu
from jax.experimental.pallas import tpu_sc as plsc

E = 8
K = 2
D = 1024
F = 2048
N = 4096
P = N * K
TILE = 256
L_ROUTED = P + E * TILE
NUM_RTILES = L_ROUTED // TILE
NUM_STILES = N // TILE

_NC = 2
_NS = 16
_NW = _NC * _NS


def _router_body(x_ref, wr_ref, meta_ref):
    lt = lax.dot_general(wr_ref[...].astype(jnp.bfloat16),
                         x_ref[...].astype(jnp.bfloat16),
                         (((1,), (1,)), ((), ())),
                         preferred_element_type=jnp.float32)
    row = lax.broadcasted_iota(jnp.int32, (E, N), 0)
    l1 = jnp.max(lt, axis=0, keepdims=True)
    i1 = jnp.min(jnp.where(lt == l1, row, E), axis=0, keepdims=True)
    masked = jnp.where(row == i1, -jnp.inf, lt)
    l2 = jnp.max(masked, axis=0, keepdims=True)
    i2 = jnp.min(jnp.where(masked == l2, row, E), axis=0, keepdims=True)
    w1 = 1.0 / (1.0 + jnp.exp(l2 - l1))
    w2 = 1.0 / (1.0 + jnp.exp(l1 - l2))
    meta_ref[0:1, :] = i1.astype(jnp.float32)
    meta_ref[1:2, :] = i2.astype(jnp.float32)
    meta_ref[2:3, :] = w1
    meta_ref[3:4, :] = w2
    meta_ref[4:5, :] = jnp.zeros((1, N), jnp.float32)
    meta_ref[5:6, :] = jnp.zeros((1, N), jnp.float32)
    meta_ref[6:7, :] = jnp.zeros((1, N), jnp.float32)
    meta_ref[7:8, :] = jnp.zeros((1, N), jnp.float32)


def _router(xf, w_router, interpret=False):
    return pl.pallas_call(
        _router_body,
        out_shape=jax.ShapeDtypeStruct((8, N), jnp.float32),
        interpret=interpret,
    )(xf, w_router)


_ROWS_PER_W = L_ROUTED // _NW
_DCHUNK = 40
_DSTEPS = _ROWS_PER_W // _DCHUNK
_DNB = 2
_DLEAD = 1


def _dispatch_body(x_hbm, map_hbm, xg_hbm, idx_v, buf_v, gsems, wsems):
    wid = lax.axis_index("s") * _NC + lax.axis_index("c")
    wbase = wid * _ROWS_PER_W
    pltpu.sync_copy(map_hbm.at[pl.ds(wbase, _ROWS_PER_W)], idx_v)

    def gather_desc(c, fire):
        b = c % _DNB
        mk = pltpu.async_copy if fire else pltpu.make_async_copy
        return mk(x_hbm.at[idx_v.at[pl.ds(c * _DCHUNK, _DCHUNK)]],
                  buf_v.at[b], gsems.at[b])

    def write_desc(c, fire):
        b = c % _DNB
        mk = pltpu.async_copy if fire else pltpu.make_async_copy
        return mk(buf_v.at[b], xg_hbm.at[pl.ds(wbase + c * _DCHUNK,
                                               _DCHUNK)], wsems.at[b])

    for c in range(_DLEAD):
        gather_desc(c, True)
    for c in range(_DSTEPS):
        gather_desc(c, False).wait()
        write_desc(c, True)
        j = c + _DLEAD
        if j < _DSTEPS:
            if j - _DNB >= 0:
                write_desc(j - _DNB, False).wait()
            gather_desc(j, True)
    for c in range(_DSTEPS - _DNB, _DSTEPS):
        write_desc(c, False).wait()


def _dispatch(xf, src_map, interpret=False):
    mesh = plsc.VectorSubcoreMesh(core_axis_name="c", subcore_axis_name="s")
    f = functools.partial(
        pl.kernel,
        out_type=jax.ShapeDtypeStruct((L_ROUTED, D), jnp.float32),
        mesh=mesh,
        scratch_types=[
            pltpu.VMEM((_ROWS_PER_W,), jnp.int32),
            pltpu.VMEM((_DNB, _DCHUNK, D), jnp.float32),
            pltpu.SemaphoreType.DMA((_DNB,)),
            pltpu.SemaphoreType.DMA((_DNB,)),
        ],
        interpret=interpret,
    )(_dispatch_body)
    return f(xf, src_map)


def _silu(x):
    return x / (1.0 + jnp.exp(-x))


def _gemm_body(sched_ref, xg_ref, g_ref, u_ref, d_ref, out_ref):
    t = pl.program_id(0)
    te = sched_ref[t]
    row0 = t * TILE

    @pl.when(row0 < sched_ref[NUM_RTILES + te])
    def _():
        xb = xg_ref[...]
        hg = lax.dot_general(xb, g_ref[0], (((1,), (1,)), ((), ())),
                             preferred_element_type=jnp.float32)
        hu = lax.dot_general(xb, u_ref[0], (((1,), (1,)), ((), ())),
                             preferred_element_type=jnp.float32)
        h = _silu(hg) * hu
        out_ref[...] = lax.dot_general(h, d_ref[0], (((1,), (1,)), ((), ())),
                                       preferred_element_type=jnp.float32)


def _gemm(sched, xg, experts_gate, experts_up, experts_down, interpret=False):
    grid_spec = pltpu.PrefetchScalarGridSpec(
        num_scalar_prefetch=1,
        grid=(NUM_RTILES,),
        in_specs=[
            pl.BlockSpec((TILE, D), lambda t, s: (t, 0)),
            pl.BlockSpec((1, F, D), lambda t, s: (s[t], 0, 0)),
            pl.BlockSpec((1, F, D), lambda t, s: (s[t], 0, 0)),
            pl.BlockSpec((1, D, F), lambda t, s: (s[t], 0, 0)),
        ],
        out_specs=pl.BlockSpec((TILE, D), lambda t, s: (t, 0)),
    )
    return pl.pallas_call(
        _gemm_body,
        grid_spec=grid_spec,
        out_shape=jax.ShapeDtypeStruct((L_ROUTED, D), jnp.float32),
        interpret=interpret,
    )(sched, xg, experts_gate, experts_up, experts_down)


def _shared_body(x_ref, g_ref, u_ref, d_ref, out_ref):
    xb = x_ref[...]
    hg = lax.dot_general(xb, g_ref[...], (((1,), (1,)), ((), ())),
                         preferred_element_type=jnp.float32)
    hu = lax.dot_general(xb, u_ref[...], (((1,), (1,)), ((), ())),
                         preferred_element_type=jnp.float32)
    h = _silu(hg) * hu
    out_ref[...] = lax.dot_general(h, d_ref[...], (((1,), (1,)), ((), ())),
                                   preferred_element_type=jnp.float32)


def _shared(xf, g, u, d, interpret=False):
    return pl.pallas_call(
        _shared_body,
        grid=(NUM_STILES,),
        in_specs=[
            pl.BlockSpec((TILE, D), lambda t: (t, 0)),
            pl.BlockSpec((F, D), lambda t: (0, 0)),
            pl.BlockSpec((F, D), lambda t: (0, 0)),
            pl.BlockSpec((D, F), lambda t: (0, 0)),
        ],
        out_specs=pl.BlockSpec((TILE, D), lambda t: (t, 0)),
        out_shape=jax.ShapeDtypeStruct((N, D), jnp.float32),
        interpret=interpret,
    )(xf, g, u, d)


_TOK_PER_W = N // _NW
_CCHUNK = 8
_CSTEPS = _TOK_PER_W // _CCHUNK


def _combine_body(eout_hbm, es_hbm, pos1_hbm, pos2_hbm, w1_hbm, w2_hbm,
                  out_hbm, p1_v, p2_v, w1_v, w2_v, r1_v, r2_v, rs_v, out_v,
                  g1sems, g2sems, gssems, wsems):
    wid = lax.axis_index("s") * _NC + lax.axis_index("c")
    wbase = wid * _TOK_PER_W
    pltpu.sync_copy(pos1_hbm.at[pl.ds(wbase, _TOK_PER_W)], p1_v)
    pltpu.sync_copy(pos2_hbm.at[pl.ds(wbase, _TOK_PER_W)], p2_v)
    pltpu.sync_copy(w1_hbm.at[pl.ds(wbase, _TOK_PER_W)], w1_v)
    pltpu.sync_copy(w2_hbm.at[pl.ds(wbase, _TOK_PER_W)], w2_v)

    def gathers(b, c, fire):
        mk = pltpu.async_copy if fire else pltpu.make_async_copy
        sl = pl.ds(pl.multiple_of(c * _CCHUNK, _CCHUNK), _CCHUNK)
        lsl = pl.ds(pl.multiple_of(wbase + c * _CCHUNK, _CCHUNK), _CCHUNK)
        return (
            mk(eout_hbm.at[p1_v.at[sl]], r1_v.at[b], g1sems.at[b]),
            mk(eout_hbm.at[p2_v.at[sl]], r2_v.at[b], g2sems.at[b]),
            mk(es_hbm.at[lsl], rs_v.at[b], gssems.at[b]),
        )

    def write_desc(b, c, fire):
        mk = pltpu.async_copy if fire else pltpu.make_async_copy
        lsl = pl.ds(pl.multiple_of(wbase + c * _CCHUNK, _CCHUNK), _CCHUNK)
        return mk(out_v.at[b], out_hbm.at[lsl], wsems.at[b])

    for b in range(2):
        gathers(b, b, True)

    def outer(k, _):
        for b in range(2):
            c = 2 * k + b
            for h in gathers(b, 0, False):
                h.wait()

            @pl.when(k > 0)
            def _():
                write_desc(b, 0, False).wait()

            def tok(i, _):
                row = c * _CCHUNK + i
                w1b = w1_v[row, :]
                w2b = w2_v[row, :]
                for col in range(D // 16):
                    sl = pl.ds(col * 16, 16)
                    out_v[b, i, sl] = (r1_v[b, i, sl] * w1b
                                       + r2_v[b, i, sl] * w2b
                                       + rs_v[b, i, sl])
                return 0

            lax.fori_loop(0, _CCHUNK, tok, 0)
            write_desc(b, c, True)

            @pl.when(k < _CSTEPS // 2 - 1)
            def _():
                gathers(b, c + 2, True)
        return 0

    lax.fori_loop(0, _CSTEPS // 2, outer, 0)
    for b in range(2):
        write_desc(b, 0, False).wait()


def _combine(eout, eshared, pos1, pos2, w1, w2, interpret=False):
    mesh = plsc.VectorSubcoreMesh(core_axis_name="c", subcore_axis_name="s")
    f = functools.partial(
        pl.kernel,
        out_type=jax.ShapeDtypeStruct((N, D), jnp.float32),
        mesh=mesh,
        scratch_types=[
            pltpu.VMEM((_TOK_PER_W,), jnp.int32),
            pltpu.VMEM((_TOK_PER_W,), jnp.int32),
            pltpu.VMEM((_TOK_PER_W, 16), jnp.float32),
            pltpu.VMEM((_TOK_PER_W, 16), jnp.float32),
            pltpu.VMEM((2, _CCHUNK, D), jnp.float32),
            pltpu.VMEM((2, _CCHUNK, D), jnp.float32),
            pltpu.VMEM((2, _CCHUNK, D), jnp.float32),
            pltpu.VMEM((2, _CCHUNK, D), jnp.float32),
            pltpu.SemaphoreType.DMA((2,)),
            pltpu.SemaphoreType.DMA((2,)),
            pltpu.SemaphoreType.DMA((2,)),
            pltpu.SemaphoreType.DMA((2,)),
        ],
        interpret=interpret,
    )(_combine_body)
    w1b = jnp.broadcast_to(w1[:, None], (N, 16))
    w2b = jnp.broadcast_to(w2[:, None], (N, 16))
    return f(eout, eshared, pos1, pos2, w1b, w2b)


def kernel(x, w_router, shared_gate, shared_up, shared_down,
           experts_gate, experts_up, experts_down):
    xf = x.reshape(N, D)

    meta = _router(xf, w_router)
    e1 = meta[0].astype(jnp.int32)
    e2 = meta[1].astype(jnp.int32)
    w1 = meta[2]
    w2 = meta[3]

    pairs_e = jnp.stack([e1, e2], axis=1).reshape(-1)
    oh = (pairs_e[:, None] == jnp.arange(E)[None, :]).astype(jnp.int32)
    cum = jnp.cumsum(oh, axis=0)
    counts = cum[-1]
    rank = jnp.sum(oh * (cum - oh), axis=1)
    pc = ((counts + TILE - 1) // TILE) * TILE
    bounds = jnp.cumsum(pc)
    pstart = bounds - pc
    dst_pair = (jnp.sum(oh * pstart[None, :], axis=1) + rank).astype(jnp.int32)
    pos1 = dst_pair[0::2]
    pos2 = dst_pair[1::2]
    src_map = jnp.zeros(L_ROUTED, jnp.int32).at[dst_pair].set(
        jnp.arange(P, dtype=jnp.int32) // K)
    t_row0 = jnp.arange(NUM_RTILES, dtype=jnp.int32) * TILE
    tile_expert = jnp.minimum(
        jnp.sum((t_row0[:, None] >= bounds[None, :]).astype(jnp.int32),
                axis=1), E - 1)
    real_end = pstart + counts
    sched = jnp.concatenate([tile_expert, real_end])

    xg = _dispatch(xf, src_map)
    eshared = _shared(xf, shared_gate, shared_up, shared_down)
    eout = _gemm(sched, xg, experts_gate, experts_up, experts_down)
    out = _combine(eout, eshared, pos1, pos2, w1, w2)
    return out.reshape(x.shape)
